# baseline (speedup 1.0000x reference)
import jax
import jax.numpy as jnp
from jax.experimental import pallas as pl
from jax.experimental.pallas import tpu as pltpu

_CH = 512
_NBUF = 8
_LA = 4


def _ring_kernel(pos_ref, out_ref, vbuf, sin, sout):
    n = pos_ref.shape[0] // _CH

    def src(k):
        return pos_ref.at[pl.ds(k * _CH, _CH)]

    def dst(k):
        return out_ref.at[pl.ds(k * _CH, _CH)]

    cin, cout = {}, {}
    for j in range(min(_LA, n)):
        cin[j] = pltpu.make_async_copy(src(j), vbuf.at[j % _NBUF], sin.at[j % _NBUF])
        cin[j].start()
    for k in range(n):
        nk = k + _LA
        if nk < n:
            if nk >= _NBUF:
                cout[nk - _NBUF].wait()
            cin[nk] = pltpu.make_async_copy(
                src(nk), vbuf.at[nk % _NBUF], sin.at[nk % _NBUF]
            )
            cin[nk].start()
        cin[k].wait()
        cout[k] = pltpu.make_async_copy(
            vbuf.at[k % _NBUF], dst(k), sout.at[k % _NBUF]
        )
        cout[k].start()
    for k in range(max(0, n - _NBUF), n):
        cout[k].wait()


def kernel(x, pos_table):
    seq_len = x.shape[1]
    d_model = pos_table.shape[1]
    return pl.pallas_call(
        _ring_kernel,
        out_shape=jax.ShapeDtypeStruct((seq_len, d_model), pos_table.dtype),
        in_specs=[pl.BlockSpec(memory_space=pl.ANY)],
        out_specs=pl.BlockSpec(memory_space=pl.ANY),
        scratch_shapes=[
            pltpu.VMEM((_NBUF, _CH, d_model), pos_table.dtype),
            pltpu.SemaphoreType.DMA((_NBUF,)),
            pltpu.SemaphoreType.DMA((_NBUF,)),
        ],
        compiler_params=pltpu.CompilerParams(
            vmem_limit_bytes=64 * 1024 * 1024,
        ),
    )(pos_table)


# speedup vs baseline: 1.0150x; 1.0150x over previous
import jax
import jax.numpy as jnp
from jax.experimental import pallas as pl
from jax.experimental.pallas import tpu as pltpu


def _embed_kernel(pos_ref, out_ref):
    out_ref[...] = pos_ref[...]


def kernel(x, pos_table):
    seq_len = x.shape[1]
    d_model = pos_table.shape[1]
    block_rows = 2048
    grid = seq_len // block_rows
    return pl.pallas_call(
        _embed_kernel,
        out_shape=jax.ShapeDtypeStruct((seq_len, d_model), pos_table.dtype),
        grid=(grid,),
        in_specs=[pl.BlockSpec((block_rows, d_model), lambda i: (i, 0))],
        out_specs=pl.BlockSpec((block_rows, d_model), lambda i: (i, 0)),
        compiler_params=pltpu.CompilerParams(
            dimension_semantics=("parallel",),
        ),
    )(pos_table)
